# baseline (speedup 1.0000x reference)
import functools

import jax
import jax.numpy as jnp
from jax import lax
from jax.experimental import pallas as pl
from jax.experimental.pallas import tpu as pltpu
from jax.experimental.pallas import tpu_sc as plsc

_VOCAB = 1000
_HIDDEN = 16
_BATCH = 4096
_SEQ = 20

_NC = 2
_NS = 16
_NW = _NC * _NS
_BW = _BATCH // _NW
_BN = 4096


@functools.cache
def _make_hidden_t():
    mesh = plsc.VectorSubcoreMesh(core_axis_name="c", subcore_axis_name="s")

    @functools.partial(
        pl.kernel,
        mesh=mesh,
        out_type=jax.ShapeDtypeStruct((_SEQ, _HIDDEN, _BATCH), jnp.float32),
        scratch_types=[
            pltpu.VMEM((_VOCAB * _HIDDEN,), jnp.float32),
            pltpu.VMEM((_SEQ * _BW,), jnp.int32),
            pltpu.VMEM((_HIDDEN, _BW), jnp.float32),
            pltpu.VMEM((_HIDDEN, _BW), jnp.float32),
            pltpu.SemaphoreType.DMA,
            pltpu.SemaphoreType.DMA,
        ],
        compiler_params=pltpu.CompilerParams(
            use_tc_tiling_on_sc=False, needs_layout_passes=False),
    )
    def _hidden_t(emb_hbm, ids_hbm, out_hbm, emb_v, ids_v, b_0, b_1, s_0, s_1):
        wid = lax.axis_index("s") * _NC + lax.axis_index("c")
        b0 = pl.multiple_of(wid * _BW, 128)
        ibase = pl.multiple_of(wid * _SEQ * _BW, 8)
        pltpu.sync_copy(emb_hbm, emb_v)
        pltpu.sync_copy(ids_hbm.at[pl.ds(ibase, _SEQ * _BW)], ids_v)
        bufs = (b_0, b_1)
        sems = (s_0, s_1)

        def fill(l, buf):
            def body(j, carry):
                off = pl.multiple_of(j * 16, 8)
                base = ids_v[pl.ds(l * _BW + off, 16)] * _HIDDEN
                for h in range(_HIDDEN):
                    buf[h, pl.ds(off, 16)] = plsc.load_gather(
                        emb_v, [base + h])
                return carry

            lax.fori_loop(0, _BW // 16, body, 0)

        def flush(l, buf, sem):
            return pltpu.make_async_copy(
                buf, out_hbm.at[l, :, pl.ds(b0, _BW)], sem)

        for l in range(_SEQ):
            buf, sem = bufs[l % 2], sems[l % 2]
            if l >= 2:
                flush(l - 2, buf, sem).wait()
            fill(l, buf)
            flush(l, buf, sem).start()
        flush(_SEQ - 2, bufs[0], sems[0]).wait()
        flush(_SEQ - 1, bufs[1], sems[1]).wait()

    return _hidden_t


def _logits_kernel(w_ref, b_ref, h_ref, out_ref):
    m = lax.dot_general(
        w_ref[...], h_ref[0],
        dimension_numbers=(((1,), (0,)), ((), ())),
        preferred_element_type=jnp.float32,
    )
    out_ref[0] = m + b_ref[...]


def _logits_t(fc_w, fc_b, hidden_t):
    return pl.pallas_call(
        _logits_kernel,
        grid=(_SEQ, _BATCH // _BN),
        in_specs=[
            pl.BlockSpec((_VOCAB, _HIDDEN), lambda l, n: (0, 0)),
            pl.BlockSpec((_VOCAB, 1), lambda l, n: (0, 0)),
            pl.BlockSpec((1, _HIDDEN, _BN), lambda l, n: (l, 0, n)),
        ],
        out_specs=pl.BlockSpec((1, _VOCAB, _BN), lambda l, n: (l, 0, n)),
        out_shape=jax.ShapeDtypeStruct((_SEQ, _VOCAB, _BATCH), jnp.float32),
    )(fc_w, fc_b.reshape(_VOCAB, 1), hidden_t)


def kernel(input_ids, embed_table, fc_w, fc_b):
    emb_flat = embed_table.reshape(_VOCAB * _HIDDEN)
    ids_w = (input_ids.T.reshape(_SEQ, _NW, _BW)
             .transpose(1, 0, 2).reshape(_SEQ * _BATCH))
    hidden_t = _make_hidden_t()(emb_flat, ids_w)
    out_t = _logits_t(fc_w, fc_b, hidden_t)
    return jnp.transpose(out_t, (2, 0, 1))


# speedup vs baseline: 1.0076x; 1.0076x over previous
import functools

import jax
import jax.numpy as jnp
from jax import lax
from jax.experimental import pallas as pl
from jax.experimental.pallas import tpu as pltpu
from jax.experimental.pallas import tpu_sc as plsc

_VOCAB = 1000
_HIDDEN = 16
_BATCH = 4096
_SEQ = 20

_NC = 2
_NS = 16
_NW = _NC * _NS
_BW = _BATCH // _NW
_BN = 2048


@functools.cache
def _make_hidden_t():
    mesh = plsc.VectorSubcoreMesh(core_axis_name="c", subcore_axis_name="s")

    @functools.partial(
        pl.kernel,
        mesh=mesh,
        out_type=jax.ShapeDtypeStruct((_SEQ, _HIDDEN, _BATCH), jnp.float32),
        scratch_types=[
            pltpu.VMEM((_VOCAB * _HIDDEN,), jnp.float32),
            pltpu.VMEM((_SEQ * _BW,), jnp.int32),
            pltpu.VMEM((_HIDDEN, _BW), jnp.float32),
            pltpu.VMEM((_HIDDEN, _BW), jnp.float32),
            pltpu.SemaphoreType.DMA,
            pltpu.SemaphoreType.DMA,
        ],
        compiler_params=pltpu.CompilerParams(
            use_tc_tiling_on_sc=False, needs_layout_passes=False),
    )
    def _hidden_t(emb_hbm, ids_hbm, out_hbm, emb_v, ids_v, b_0, b_1, s_0, s_1):
        wid = lax.axis_index("s") * _NC + lax.axis_index("c")
        b0 = pl.multiple_of(wid * _BW, 128)
        ibase = pl.multiple_of(wid * _SEQ * _BW, 8)
        pltpu.sync_copy(emb_hbm, emb_v)
        pltpu.sync_copy(ids_hbm.at[pl.ds(ibase, _SEQ * _BW)], ids_v)
        bufs = (b_0, b_1)
        sems = (s_0, s_1)

        def fill(l, buf):
            def body(j, carry):
                off = pl.multiple_of(j * 16, 8)
                base = ids_v[pl.ds(l * _BW + off, 16)] * _HIDDEN
                for h in range(_HIDDEN):
                    buf[h, pl.ds(off, 16)] = plsc.load_gather(
                        emb_v, [base + h])
                return carry

            lax.fori_loop(0, _BW // 16, body, 0)

        def flush(l, buf, sem):
            return pltpu.make_async_copy(
                buf, out_hbm.at[l, :, pl.ds(b0, _BW)], sem)

        for l in range(_SEQ):
            buf, sem = bufs[l % 2], sems[l % 2]
            if l >= 2:
                flush(l - 2, buf, sem).wait()
            fill(l, buf)
            flush(l, buf, sem).start()
        flush(_SEQ - 2, bufs[0], sems[0]).wait()
        flush(_SEQ - 1, bufs[1], sems[1]).wait()

    return _hidden_t


def _logits_kernel(w_ref, b_ref, h_ref, out_ref):
    m = lax.dot_general(
        w_ref[...], h_ref[0],
        dimension_numbers=(((1,), (0,)), ((), ())),
        preferred_element_type=jnp.float32,
    )
    out_ref[0] = m + b_ref[...]


def _logits_t(fc_w, fc_b, hidden_t):
    return pl.pallas_call(
        _logits_kernel,
        grid=(_SEQ, _BATCH // _BN),
        in_specs=[
            pl.BlockSpec((_VOCAB, _HIDDEN), lambda l, n: (0, 0)),
            pl.BlockSpec((_VOCAB, 1), lambda l, n: (0, 0)),
            pl.BlockSpec((1, _HIDDEN, _BN), lambda l, n: (l, 0, n)),
        ],
        out_specs=pl.BlockSpec((1, _VOCAB, _BN), lambda l, n: (l, 0, n)),
        out_shape=jax.ShapeDtypeStruct((_SEQ, _VOCAB, _BATCH), jnp.float32),
    )(fc_w, fc_b.reshape(_VOCAB, 1), hidden_t)


def kernel(input_ids, embed_table, fc_w, fc_b):
    emb_flat = embed_table.reshape(_VOCAB * _HIDDEN)
    ids_w = (input_ids.T.reshape(_SEQ, _NW, _BW)
             .transpose(1, 0, 2).reshape(_SEQ * _BATCH))
    hidden_t = _make_hidden_t()(emb_flat, ids_w)
    out_t = _logits_t(fc_w, fc_b, hidden_t)
    return jnp.transpose(out_t, (2, 0, 1))


# speedup vs baseline: 1.0698x; 1.0617x over previous
import functools

import jax
import jax.numpy as jnp
from jax import lax
from jax.experimental import pallas as pl
from jax.experimental.pallas import tpu as pltpu
from jax.experimental.pallas import tpu_sc as plsc

_VOCAB = 1000
_HIDDEN = 16
_BATCH = 4096
_SEQ = 20

_NC = 2
_NS = 16
_NW = _NC * _NS
_BW = _BATCH // _NW
_BN = 2048


@functools.cache
def _make_hidden_t():
    mesh = plsc.VectorSubcoreMesh(core_axis_name="c", subcore_axis_name="s")

    @functools.partial(
        pl.kernel,
        mesh=mesh,
        out_type=jax.ShapeDtypeStruct((_SEQ, _HIDDEN, _BATCH), jnp.float32),
        scratch_types=[
            pltpu.VMEM((_VOCAB * _HIDDEN,), jnp.float32),
            pltpu.VMEM((_SEQ * _BW,), jnp.int32),
            pltpu.VMEM((_HIDDEN, _BW), jnp.float32),
            pltpu.VMEM((_HIDDEN, _BW), jnp.float32),
            pltpu.SemaphoreType.DMA,
            pltpu.SemaphoreType.DMA,
        ],
        compiler_params=pltpu.CompilerParams(
            use_tc_tiling_on_sc=False, needs_layout_passes=False),
    )
    def _hidden_t(emb_hbm, ids_hbm, out_hbm, emb_v, ids_v, b_0, b_1, s_0, s_1):
        wid = lax.axis_index("s") * _NC + lax.axis_index("c")
        b0 = pl.multiple_of(wid * _BW, 128)
        ibase = pl.multiple_of(wid * _SEQ * _BW, 8)
        pltpu.sync_copy(emb_hbm, emb_v)
        pltpu.sync_copy(ids_hbm.at[pl.ds(ibase, _SEQ * _BW)], ids_v)
        bufs = (b_0, b_1)
        sems = (s_0, s_1)
        lane_l = jnp.arange(16, dtype=jnp.int32) * _SEQ

        def fill(l, buf):
            @plsc.parallel_loop(0, _BW // 16, unroll=2)
            def body(j):
                off = pl.multiple_of(j * 16, 8)
                ids16 = plsc.load_gather(ids_v, [lane_l + (off * _SEQ + l)])
                base = ids16 * _HIDDEN
                for h in range(_HIDDEN):
                    buf[h, pl.ds(off, 16)] = plsc.load_gather(
                        emb_v, [base + h])

        def flush(l, buf, sem):
            return pltpu.make_async_copy(
                buf, out_hbm.at[l, :, pl.ds(b0, _BW)], sem)

        for l in range(_SEQ):
            buf, sem = bufs[l % 2], sems[l % 2]
            if l >= 2:
                flush(l - 2, buf, sem).wait()
            fill(l, buf)
            flush(l, buf, sem).start()
        flush(_SEQ - 2, bufs[0], sems[0]).wait()
        flush(_SEQ - 1, bufs[1], sems[1]).wait()

    return _hidden_t


def _logits_kernel(w_ref, b_ref, h_ref, out_ref):
    m = lax.dot_general(
        w_ref[...], h_ref[0],
        dimension_numbers=(((1,), (0,)), ((), ())),
        preferred_element_type=jnp.float32,
    )
    out_ref[0] = m + b_ref[...]


def _logits_t(fc_w, fc_b, hidden_t):
    return pl.pallas_call(
        _logits_kernel,
        grid=(_SEQ, _BATCH // _BN),
        in_specs=[
            pl.BlockSpec((_VOCAB, _HIDDEN), lambda l, n: (0, 0)),
            pl.BlockSpec((_VOCAB, 1), lambda l, n: (0, 0)),
            pl.BlockSpec((1, _HIDDEN, _BN), lambda l, n: (l, 0, n)),
        ],
        out_specs=pl.BlockSpec((1, _VOCAB, _BN), lambda l, n: (l, 0, n)),
        out_shape=jax.ShapeDtypeStruct((_SEQ, _VOCAB, _BATCH), jnp.float32),
    )(fc_w, fc_b.reshape(_VOCAB, 1), hidden_t)


def kernel(input_ids, embed_table, fc_w, fc_b):
    emb_flat = embed_table.reshape(_VOCAB * _HIDDEN)
    ids_flat = input_ids.reshape(_BATCH * _SEQ)
    hidden_t = _make_hidden_t()(emb_flat, ids_flat)
    out_t = _logits_t(fc_w, fc_b, hidden_t)
    return jnp.transpose(out_t, (2, 0, 1))


# speedup vs baseline: 1.0705x; 1.0006x over previous
import functools

import jax
import jax.numpy as jnp
from jax import lax
from jax.experimental import pallas as pl
from jax.experimental.pallas import tpu as pltpu
from jax.experimental.pallas import tpu_sc as plsc

_VOCAB = 1000
_HIDDEN = 16
_BATCH = 4096
_SEQ = 20

_NC = 2
_NS = 16
_NW = _NC * _NS
_BW = _BATCH // _NW
_BN = 2048


@functools.cache
def _make_hidden_t():
    mesh = plsc.VectorSubcoreMesh(core_axis_name="c", subcore_axis_name="s")

    @functools.partial(
        pl.kernel,
        mesh=mesh,
        out_type=jax.ShapeDtypeStruct((_SEQ, _HIDDEN, _BATCH), jnp.float32),
        scratch_types=[
            pltpu.VMEM((_VOCAB * _HIDDEN,), jnp.float32),
            pltpu.VMEM((_SEQ * _BW,), jnp.int32),
            pltpu.VMEM((_HIDDEN, _BW), jnp.float32),
            pltpu.VMEM((_HIDDEN, _BW), jnp.float32),
            pltpu.SemaphoreType.DMA,
            pltpu.SemaphoreType.DMA,
        ],
        compiler_params=pltpu.CompilerParams(
            use_tc_tiling_on_sc=False, needs_layout_passes=False),
    )
    def _hidden_t(emb_hbm, ids_hbm, out_hbm, emb_v, ids_v, b_0, b_1, s_0, s_1):
        wid = lax.axis_index("s") * _NC + lax.axis_index("c")
        b0 = pl.multiple_of(wid * _BW, 128)
        ibase = pl.multiple_of(wid * _SEQ * _BW, 8)
        pltpu.sync_copy(emb_hbm, emb_v)
        pltpu.sync_copy(ids_hbm.at[pl.ds(ibase, _SEQ * _BW)], ids_v)
        bufs = (b_0, b_1)
        sems = (s_0, s_1)
        lane_l = jnp.arange(16, dtype=jnp.int32) * _SEQ

        def fill(l, buf):
            @plsc.parallel_loop(0, _BW // 16, unroll=4)
            def body(j):
                off = pl.multiple_of(j * 16, 8)
                ids16 = plsc.load_gather(ids_v, [lane_l + (off * _SEQ + l)])
                base = ids16 * _HIDDEN
                for h in range(_HIDDEN):
                    buf[h, pl.ds(off, 16)] = plsc.load_gather(
                        emb_v, [base + h])

        def flush(l, buf, sem):
            return pltpu.make_async_copy(
                buf, out_hbm.at[l, :, pl.ds(b0, _BW)], sem)

        for l in range(_SEQ):
            buf, sem = bufs[l % 2], sems[l % 2]
            if l >= 2:
                flush(l - 2, buf, sem).wait()
            fill(l, buf)
            flush(l, buf, sem).start()
        flush(_SEQ - 2, bufs[0], sems[0]).wait()
        flush(_SEQ - 1, bufs[1], sems[1]).wait()

    return _hidden_t


def _logits_kernel(w_ref, b_ref, h_ref, out_ref):
    m = lax.dot_general(
        w_ref[...], h_ref[0],
        dimension_numbers=(((1,), (0,)), ((), ())),
        preferred_element_type=jnp.float32,
    )
    out_ref[0] = m + b_ref[...]


def _logits_t(fc_w, fc_b, hidden_t):
    return pl.pallas_call(
        _logits_kernel,
        grid=(_SEQ, _BATCH // _BN),
        in_specs=[
            pl.BlockSpec((_VOCAB, _HIDDEN), lambda l, n: (0, 0)),
            pl.BlockSpec((_VOCAB, 1), lambda l, n: (0, 0)),
            pl.BlockSpec((1, _HIDDEN, _BN), lambda l, n: (l, 0, n)),
        ],
        out_specs=pl.BlockSpec((1, _VOCAB, _BN), lambda l, n: (l, 0, n)),
        out_shape=jax.ShapeDtypeStruct((_SEQ, _VOCAB, _BATCH), jnp.float32),
    )(fc_w, fc_b.reshape(_VOCAB, 1), hidden_t)


def kernel(input_ids, embed_table, fc_w, fc_b):
    emb_flat = embed_table.reshape(_VOCAB * _HIDDEN)
    ids_flat = input_ids.reshape(_BATCH * _SEQ)
    hidden_t = _make_hidden_t()(emb_flat, ids_flat)
    out_t = _logits_t(fc_w, fc_b, hidden_t)
    return jnp.transpose(out_t, (2, 0, 1))


# speedup vs baseline: 1.1199x; 1.0462x over previous
import functools

import jax
import jax.numpy as jnp
from jax import lax
from jax.experimental import pallas as pl
from jax.experimental.pallas import tpu as pltpu
from jax.experimental.pallas import tpu_sc as plsc

_VOCAB = 1000
_HIDDEN = 16
_BATCH = 4096
_SEQ = 20

_NC = 2
_NS = 16
_NW = _NC * _NS
_BW = _BATCH // _NW
_BN = 2048


@functools.cache
def _make_hidden_t():
    mesh = plsc.VectorSubcoreMesh(core_axis_name="c", subcore_axis_name="s")

    @functools.partial(
        pl.kernel,
        mesh=mesh,
        out_type=jax.ShapeDtypeStruct((_SEQ, _HIDDEN, _BATCH), jnp.float32),
        scratch_types=[
            pltpu.VMEM((_VOCAB * _HIDDEN,), jnp.float32),
            pltpu.VMEM((_SEQ * _BW,), jnp.int32),
            pltpu.VMEM((_HIDDEN, _BW), jnp.float32),
            pltpu.VMEM((_HIDDEN, _BW), jnp.float32),
            pltpu.SemaphoreType.DMA,
            pltpu.SemaphoreType.DMA,
        ],
        compiler_params=pltpu.CompilerParams(
            needs_layout_passes=False),
    )
    def _hidden_t(emb_hbm, ids_hbm, out_hbm, emb_v, ids_v, b_0, b_1, s_0, s_1):
        wid = lax.axis_index("s") * _NC + lax.axis_index("c")
        b0 = pl.multiple_of(wid * _BW, 128)
        ibase = pl.multiple_of(wid * _SEQ * _BW, 8)
        pltpu.sync_copy(emb_hbm, emb_v)
        pltpu.sync_copy(ids_hbm.at[pl.ds(ibase, _SEQ * _BW)], ids_v)
        bufs = (b_0, b_1)
        sems = (s_0, s_1)
        lane_l = jnp.arange(16, dtype=jnp.int32) * _SEQ

        def fill(l, buf):
            @plsc.parallel_loop(0, _BW // 16, unroll=4)
            def body(j):
                off = pl.multiple_of(j * 16, 8)
                ids16 = plsc.load_gather(ids_v, [lane_l + (off * _SEQ + l)])
                base = ids16 * _HIDDEN
                for h in range(_HIDDEN):
                    buf[h, pl.ds(off, 16)] = plsc.load_gather(
                        emb_v, [base + h])

        def flush(l, buf, sem):
            return pltpu.make_async_copy(
                buf, out_hbm.at[l, :, pl.ds(b0, _BW)], sem)

        for l in range(_SEQ):
            buf, sem = bufs[l % 2], sems[l % 2]
            if l >= 2:
                flush(l - 2, buf, sem).wait()
            fill(l, buf)
            flush(l, buf, sem).start()
        flush(_SEQ - 2, bufs[0], sems[0]).wait()
        flush(_SEQ - 1, bufs[1], sems[1]).wait()

    return _hidden_t


def _logits_kernel(w_ref, b_ref, h_ref, out_ref):
    m = lax.dot_general(
        w_ref[...], h_ref[0],
        dimension_numbers=(((1,), (0,)), ((), ())),
        preferred_element_type=jnp.float32,
    )
    out_ref[0] = m + b_ref[...]


def _logits_t(fc_w, fc_b, hidden_t):
    return pl.pallas_call(
        _logits_kernel,
        grid=(_SEQ, _BATCH // _BN),
        in_specs=[
            pl.BlockSpec((_VOCAB, _HIDDEN), lambda l, n: (0, 0)),
            pl.BlockSpec((_VOCAB, 1), lambda l, n: (0, 0)),
            pl.BlockSpec((1, _HIDDEN, _BN), lambda l, n: (l, 0, n)),
        ],
        out_specs=pl.BlockSpec((1, _VOCAB, _BN), lambda l, n: (l, 0, n)),
        out_shape=jax.ShapeDtypeStruct((_SEQ, _VOCAB, _BATCH), jnp.float32),
    )(fc_w, fc_b.reshape(_VOCAB, 1), hidden_t)


def kernel(input_ids, embed_table, fc_w, fc_b):
    emb_flat = embed_table.reshape(_VOCAB * _HIDDEN)
    ids_flat = input_ids.reshape(_BATCH * _SEQ)
    hidden_t = _make_hidden_t()(emb_flat, ids_flat)
    out_t = _logits_t(fc_w, fc_b, hidden_t)
    return jnp.transpose(out_t, (2, 0, 1))
